# baseline (speedup 1.0000x reference)
import functools

import jax
import jax.numpy as jnp
from jax import lax
from jax.experimental import pallas as pl
from jax.experimental.pallas import tpu as pltpu
from jax.experimental.pallas import tpu_sc as plsc

_NC, _NS, _L = 2, 16, 16
_NW = _NC * _NS
_BC = 8


def _gather_temporal(temporal, n_patches):
    P, D = temporal.shape
    rows_w = P // _NW
    npat = jnp.full((_L,), n_patches, dtype=jnp.int32)
    mesh = plsc.VectorSubcoreMesh(core_axis_name="c", subcore_axis_name="s")

    @functools.partial(
        pl.kernel,
        out_type=jax.ShapeDtypeStruct((P, D), jnp.float32),
        mesh=mesh,
        scratch_types=[
            pltpu.VMEM((rows_w, D), jnp.float32),
            pltpu.VMEM((rows_w,), jnp.int32),
            pltpu.VMEM((_L,), jnp.int32),
            pltpu.SemaphoreType.DMA,
        ],
    )
    def k(t_hbm, np_hbm, o_hbm, rows_v, idx_v, npv, sem):
        wid = lax.axis_index("s") * _NC + lax.axis_index("c")
        base = wid * rows_w
        pltpu.sync_copy(np_hbm, npv)
        off = npv[...] - P
        for g in range(rows_w // _L):
            ids = lax.iota(jnp.int32, _L) + base + g * _L + off
            idx_v[pl.ds(g * _L, _L)] = jnp.clip(ids, 0, P - 1)
        pltpu.async_copy(t_hbm.at[idx_v], rows_v, sem).wait()
        pltpu.sync_copy(rows_v, o_hbm.at[pl.ds(base, rows_w)])

    return k(temporal, npat)


def _bcast_body(s_ref, t_ref, o_ref):
    c = pl.program_id(0)
    s = s_ref[pl.ds(c * _BC, _BC), :]
    o_ref[...] = s[:, None, :] + t_ref[...][None, :, :]


def kernel(spatial, temporal, n_patches):
    C, D = spatial.shape
    P, _ = temporal.shape
    t_rows = _gather_temporal(temporal, n_patches)
    out = pl.pallas_call(
        _bcast_body,
        grid=(C // _BC,),
        in_specs=[
            pl.BlockSpec((C, D), lambda c: (0, 0)),
            pl.BlockSpec((P, D), lambda c: (0, 0)),
        ],
        out_specs=pl.BlockSpec((_BC, P, D), lambda c: (c, 0, 0)),
        out_shape=jax.ShapeDtypeStruct((C, P, D), jnp.float32),
    )(spatial, t_rows)
    return out.reshape(1, C * P, D)
